# baseline (speedup 1.0000x reference)
import jax
import jax.numpy as jnp
from jax.experimental import pallas as pl

_IN = 32
_OUT = 32
_TILE = 512


def _graphconv_body(a_ref, f_all_ref, f_tile_ref, w_ref, b_ref, o_ref):
    a = a_ref[0]
    f = f_all_ref[0]
    agg = jnp.dot(a, f, preferred_element_type=jnp.float32)
    ft = f_tile_ref[0]
    w1 = w_ref[:_IN, :]
    w2 = w_ref[_IN:, :]
    out = (jnp.dot(ft, w1, preferred_element_type=jnp.float32)
           + jnp.dot(agg, w2, preferred_element_type=jnp.float32)
           + b_ref[...])
    o_ref[0] = jnp.maximum(out, 0.0)


def kernel(features, A, weight, bias):
    B, N, I, IN = features.shape
    OUT = weight.shape[1]
    f2d = features.reshape(B, N * I, IN)
    bias2d = bias.reshape(1, OUT)

    grid = (B, N // _TILE)
    out = pl.pallas_call(
        _graphconv_body,
        grid=grid,
        in_specs=[
            pl.BlockSpec((1, _TILE, N), lambda b, i: (b, i, 0)),
            pl.BlockSpec((1, N, IN), lambda b, i: (b, 0, 0)),
            pl.BlockSpec((1, _TILE, IN), lambda b, i: (b, i, 0)),
            pl.BlockSpec((weight.shape[0], OUT), lambda b, i: (0, 0)),
            pl.BlockSpec((1, OUT), lambda b, i: (0, 0)),
        ],
        out_specs=pl.BlockSpec((1, _TILE, OUT), lambda b, i: (b, i, 0)),
        out_shape=jax.ShapeDtypeStruct((B, N, OUT), jnp.float32),
    )(A, f2d, f2d, weight, bias2d)
    return out.reshape(B, N, I, OUT)
